# Initial kernel scaffold; baseline (speedup 1.0000x reference)
import functools

import jax
import jax.numpy as jnp
from jax import lax
from jax.experimental import pallas as pl
from jax.experimental.pallas import tpu as pltpu
from jax.experimental.pallas import tpu_sc as plsc

NN = 10000
NP = 10080
EE = 320000
EP = 330240
BN = 1008
AUG = 144
SB = 80
KSLAB = 2064

_F32 = jnp.float32
_I32 = jnp.int32


def _tc1_body(des, tweet, nump, catp, newf, Wd, bd, Wt, bt, Wn, bn, Wc, bc,
              Wf, bf, Wi, bi, W1, as1, ad1, haug, al):
    lr = lambda v: jnp.where(v >= 0, v, 0.01 * v)
    d = lr(jnp.dot(des[...], Wd[...], preferred_element_type=_F32) + bd[...])
    t = lr(jnp.dot(tweet[...], Wt[...], preferred_element_type=_F32) + bt[...])
    n = lr(jnp.dot(nump[...], Wn[...], preferred_element_type=_F32) + bn[...])
    c = lr(jnp.dot(catp[...], Wc[...], preferred_element_type=_F32) + bc[...])
    f = lr(jnp.dot(newf[...], Wf[...], preferred_element_type=_F32) + bf[...])
    Wi_ = Wi[...]
    acc = jnp.dot(d, Wi_[0:28, :], preferred_element_type=_F32)
    acc += jnp.dot(t, Wi_[28:64, :], preferred_element_type=_F32)
    acc += jnp.dot(n, Wi_[64:76, :], preferred_element_type=_F32)
    acc += jnp.dot(c, Wi_[76:116, :], preferred_element_type=_F32)
    acc += jnp.dot(f, Wi_[116:128, :], preferred_element_type=_F32)
    x = lr(acc + bi[...])
    onecol = jnp.concatenate(
        [jnp.ones((BN, 1), _F32), jnp.zeros((BN, AUG - 129), _F32)], axis=1)
    for hh in range(4):
        hcol = jnp.dot(x, W1[...][:, hh * 128:(hh + 1) * 128],
                       preferred_element_type=_F32)
        haug[hh, :, 0:128] = hcol
        haug[hh, :, 128:AUG] = onecol
        al[hh, :] = jnp.sum(hcol * as1[0, hh, :][None, :], axis=1)
        al[4 + hh, :] = jnp.sum(hcol * ad1[0, hh, :][None, :], axis=1)


def _run_tc1(des, tweet, nump, catp, newf, Wd, bd, Wt, bt, Wn, bn, Wc, bc,
             Wf, bf, Wi, bi, W1, as1, ad1):
    full = lambda a: pl.BlockSpec(a.shape, lambda i: tuple(0 for _ in a.shape))
    row = lambda w: pl.BlockSpec((BN, w), lambda i: (i, 0))
    args = (des, tweet, nump, catp, newf, Wd, bd, Wt, bt, Wn, bn, Wc, bc,
            Wf, bf, Wi, bi, W1, as1, ad1)
    in_specs = [row(768), row(768), row(7), row(11), row(1)] + \
        [full(a) for a in args[5:]]
    haug, al = pl.pallas_call(
        _tc1_body,
        grid=(NP // BN,),
        in_specs=in_specs,
        out_specs=[pl.BlockSpec((4, BN, AUG), lambda i: (0, i, 0)),
                   pl.BlockSpec((8, BN), lambda i: (0, i))],
        out_shape=[jax.ShapeDtypeStruct((4, NP, AUG), _F32),
                   jax.ShapeDtypeStruct((8, NP), _F32)],
    )(*args)
    return haug, al


def _tc2_body(U1, bg1, W2, as2, ad2, haug, al):
    W2_ = W2[...]
    acc = None
    for hh in range(4):
        den = jnp.maximum(U1[hh, :, 128:129], 1e-30)
        x1h = U1[hh, :, 0:128] / den + bg1[...][hh * 128:(hh + 1) * 128][None, :]
        part = jnp.dot(x1h, W2_[hh * 128:(hh + 1) * 128, :],
                       preferred_element_type=_F32)
        acc = part if acc is None else acc + part
    haug[:, 0:128] = acc
    haug[:, 128:AUG] = jnp.concatenate(
        [jnp.ones((BN, 1), _F32), jnp.zeros((BN, AUG - 129), _F32)], axis=1)
    al[0, :] = jnp.sum(acc * as2[0, 0, :][None, :], axis=1)
    al[1, :] = jnp.sum(acc * ad2[0, 0, :][None, :], axis=1)


def _run_tc2(U1, bg1, W2, as2, ad2):
    full = lambda a: pl.BlockSpec(a.shape, lambda i: tuple(0 for _ in a.shape))
    haug, al = pl.pallas_call(
        _tc2_body,
        grid=(NP // BN,),
        in_specs=[pl.BlockSpec((4, BN, AUG), lambda i: (0, i, 0)),
                  full(bg1), full(W2), full(as2), full(ad2)],
        out_specs=[pl.BlockSpec((BN, AUG), lambda i: (i, 0)),
                   pl.BlockSpec((2, BN), lambda i: (0, i))],
        out_shape=[jax.ShapeDtypeStruct((NP, AUG), _F32),
                   jax.ShapeDtypeStruct((2, NP), _F32)],
    )(U1, bg1, W2, as2, ad2)
    return haug, al


def _tc3_body(U2, bg2, Wo1, bo1, Wo2, bo2, out):
    lr = lambda v: jnp.where(v >= 0, v, 0.01 * v)
    Us = U2[0, :, :] + U2[1, :, :]
    den = jnp.maximum(Us[:, 128:129], 1e-30)
    x2 = Us[:, 0:128] / den + bg2[...][None, :]
    y = lr(jnp.dot(x2, Wo1[...], preferred_element_type=_F32) + bo1[...])
    out[...] = jnp.dot(y, Wo2[...], preferred_element_type=_F32) + bo2[...]


def _run_tc3(U2, bg2, Wo1, bo1, Wo2, bo2):
    full = lambda a: pl.BlockSpec(a.shape, lambda i: tuple(0 for _ in a.shape))
    out = pl.pallas_call(
        _tc3_body,
        grid=(NP // BN,),
        in_specs=[pl.BlockSpec((2, BN, AUG), lambda i: (0, i, 0)),
                  full(bg2), full(Wo1), full(bo1), full(Wo2), full(bo2)],
        out_specs=pl.BlockSpec((BN, 2), lambda i: (i, 0)),
        out_shape=jax.ShapeDtypeStruct((NP, 2), _F32),
    )(U2, bg2, Wo1, bo1, Wo2, bo2)
    return out


def _sc_exa_body(H, src_h, dst_h, al_h, ex_h, als_v, ald_v, sbuf, dbuf, exbuf):
    c = lax.axis_index("c")
    s = lax.axis_index("s")
    iota = lax.iota(_I32, 16)
    per_tile = H * EP // 32
    wstart = (c * 16 + s) * per_tile
    hh = wstart // EP
    e0 = wstart - hh * EP
    pltpu.sync_copy(al_h.at[pl.ds(hh * NP, NP)], als_v)
    pltpu.sync_copy(al_h.at[pl.ds((H + hh) * NP, NP)], ald_v)

    def table_max(ref):
        def mbody(i, acc):
            v = plsc.load_gather(ref, [i * 16 + iota])
            return jnp.maximum(acc, v)
        acc = lax.fori_loop(0, NP // 16, mbody, jnp.full((16,), -1e30, _F32))
        return lax.reduce_max(acc, (0,))

    M = table_max(als_v) + table_max(ald_v)

    def chunk(ci, _):
        base = e0 + ci * KSLAB
        pltpu.sync_copy(src_h.at[pl.ds(base, KSLAB)], sbuf)
        pltpu.sync_copy(dst_h.at[pl.ds(base, KSLAB)], dbuf)

        def grp(j, _):
            w = j * 16 + iota
            s16 = plsc.load_gather(sbuf, [w])
            d16 = plsc.load_gather(dbuf, [w])
            a = plsc.load_gather(als_v, [s16]) + plsc.load_gather(ald_v, [d16])
            a = jnp.where(a >= 0, a, 0.2 * a)
            plsc.store_scatter(exbuf, [w], jnp.exp(a - M))
            return 0

        lax.fori_loop(0, KSLAB // 16, grp, 0)
        pltpu.sync_copy(exbuf, ex_h.at[pl.ds(hh * EP + base, KSLAB)])
        return 0

    lax.fori_loop(0, per_tile // KSLAB, chunk, 0)


def _run_sc_exa(H, src, dst, al_flat):
    mesh = plsc.VectorSubcoreMesh(core_axis_name="c", subcore_axis_name="s")
    k = pl.kernel(
        functools.partial(_sc_exa_body, H),
        out_type=jax.ShapeDtypeStruct((H * EP,), _F32),
        mesh=mesh,
        scratch_types=[
            pltpu.VMEM((NP,), _F32),
            pltpu.VMEM((NP,), _F32),
            pltpu.VMEM((KSLAB,), _I32),
            pltpu.VMEM((KSLAB,), _I32),
            pltpu.VMEM((KSLAB,), _F32),
        ],
    )
    return k(src, dst, al_flat)


def _sc_agg_body(H, src_h, dst_h, ex_h, haug_h, z_h, U_h,
                 sbuf, dbuf2, exbuf, idxbuf, rows, gsem):
    c = lax.axis_index("c")
    s = lax.axis_index("s")
    iota = lax.iota(_I32, 16)
    n_phase = 2 if H == 4 else 1
    per_tile = EP // 16 if H == 4 else EP // 32
    nslab = per_tile // KSLAB
    zrows = NP // 16

    def scoped(U_sh):
        for ph in range(n_phase):
            if H == 4:
                hh = c * n_phase + ph
                e0 = s * per_tile
            else:
                hh = jnp.int32(0)
                e0 = c * (EP // 2) + s * per_tile
            upart = c * n_phase + ph
            pltpu.sync_copy(z_h.at[pl.ds(s * zrows, zrows)],
                            U_sh.at[pl.ds(s * zrows, zrows)])
            plsc.subcore_barrier()

            def slab(ci, _):
                base = e0 + ci * KSLAB
                pltpu.sync_copy(src_h.at[pl.ds(base, KSLAB)], sbuf)
                pltpu.sync_copy(ex_h.at[pl.ds(hh * EP + base, KSLAB)], exbuf)

                def adj(j, _):
                    w = j * 16 + iota
                    s16 = plsc.load_gather(sbuf, [w])
                    plsc.store_scatter(idxbuf, [w], s16 + hh * NP)
                    return 0

                lax.fori_loop(0, KSLAB // 16, adj, 0)

                def sub(k, _):
                    pltpu.async_copy(
                        haug_h.at[idxbuf.at[pl.ds(k * SB, SB)]],
                        rows, gsem).wait()

                    def scale(e, _):
                        ev = plsc.load_gather(exbuf, [iota * 0 + (k * SB + e)])
                        row = iota * 0 + e
                        for j in range(AUG // 16):
                            col = j * 16 + iota
                            v = plsc.load_gather(rows, [row, col])
                            plsc.store_scatter(rows, [row, col], v * ev)
                        return 0

                    lax.fori_loop(0, SB, scale, 0)
                    pltpu.sync_copy(dst_h.at[pl.ds(base + k * SB, SB)],
                                    dbuf2.at[0])
                    pltpu.sync_copy(rows, U_sh.at[dbuf2.at[0]], add=True)
                    return 0

                lax.fori_loop(0, KSLAB // SB, sub, 0)
                return 0

            lax.fori_loop(0, nslab, slab, 0)
            plsc.subcore_barrier()
            pltpu.sync_copy(U_sh.at[pl.ds(s * zrows, zrows)],
                            U_h.at[pl.ds(upart * NP + s * zrows, zrows)])
            plsc.subcore_barrier()

    pl.run_scoped(scoped, plsc.MemoryRef((NP, AUG), _F32, pltpu.VMEM_SHARED))


def _run_sc_agg(H, src, dst, ex, haug_flat, zeros_h):
    mesh = plsc.VectorSubcoreMesh(core_axis_name="c", subcore_axis_name="s")
    nout = 4 if H == 4 else 2
    k = pl.kernel(
        functools.partial(_sc_agg_body, H),
        out_type=jax.ShapeDtypeStruct((nout * NP, AUG), _F32),
        mesh=mesh,
        scratch_types=[
            pltpu.VMEM((KSLAB,), _I32),
            pltpu.VMEM((1, SB), _I32),
            pltpu.VMEM((KSLAB,), _F32),
            pltpu.VMEM((KSLAB,), _I32),
            pltpu.VMEM((SB, AUG), _F32),
            pltpu.SemaphoreType.DMA,
        ],
    )
    return k(src, dst, ex, haug_flat, zeros_h)


def kernel(des, tweet, num_prop, cat_prop, new_feature, edge_index,
           Wd, bd, Wt, bt, Wn, bn, Wc, bc, Wf, bf, Wi, bi,
           W1, as1, ad1, bg1, W2, as2, ad2, bg2, Wo1, bo1, Wo2, bo2):
    padn = lambda a: jnp.pad(a, ((0, NP - NN), (0, 0)))
    des_p = padn(des)
    tweet_p = padn(tweet)
    nump_p = padn(num_prop)
    catp_p = padn(cat_prop)
    newf_p = padn(new_feature)

    ei = edge_index.astype(_I32)
    loop = jnp.arange(NN, dtype=_I32)
    fill = jnp.full((EP - EE - NN,), NN, _I32)
    src = jnp.concatenate([ei[0], loop, fill])
    dst = jnp.concatenate([ei[1], loop, fill])
    zeros_h = jnp.zeros((NP, AUG), _F32)

    haug1, al1 = _run_tc1(des_p, tweet_p, nump_p, catp_p, newf_p,
                          Wd, bd, Wt, bt, Wn, bn, Wc, bc, Wf, bf,
                          Wi, bi, W1, as1, ad1)
    ex1 = _run_sc_exa(4, src, dst, al1.reshape(8 * NP))
    U1 = _run_sc_agg(4, src, dst, ex1, haug1.reshape(4 * NP, AUG), zeros_h)
    haug2, al2 = _run_tc2(U1.reshape(4, NP, AUG), bg1, W2, as2, ad2)
    ex2 = _run_sc_exa(1, src, dst, al2.reshape(2 * NP))
    U2 = _run_sc_agg(1, src, dst, ex2, haug2, zeros_h)
    out = _run_tc3(U2.reshape(2, NP, AUG), bg2, Wo1, bo1, Wo2, bo2)
    return out[:NN]


# speedup vs baseline: 9.4364x; 9.4364x over previous
import functools

import jax
import jax.numpy as jnp
from jax import lax
from jax.experimental import pallas as pl
from jax.experimental.pallas import tpu as pltpu
from jax.experimental.pallas import tpu_sc as plsc

NN = 10000
NP = 10240
EE = 320000
EP = 330240
BN = 1024
CF = 128
SB = 80
KSLAB = 3440

_F32 = jnp.float32
_I32 = jnp.int32


def _tc1_body(des, tweet, nump, catp, newf, Wd, bd, Wt, bt, Wn, bn, Wc, bc,
              Wf, bf, Wi, bi, W1, as1, ad1, haug, al):
    lr = lambda v: jnp.where(v >= 0, v, 0.01 * v)
    d = lr(jnp.dot(des[...], Wd[...], preferred_element_type=_F32) + bd[...])
    t = lr(jnp.dot(tweet[...], Wt[...], preferred_element_type=_F32) + bt[...])
    n = lr(jnp.dot(nump[...], Wn[...], preferred_element_type=_F32) + bn[...])
    c = lr(jnp.dot(catp[...], Wc[...], preferred_element_type=_F32) + bc[...])
    f = lr(jnp.dot(newf[...], Wf[...], preferred_element_type=_F32) + bf[...])
    Wi_ = Wi[...]
    acc = jnp.dot(d, Wi_[0:28, :], preferred_element_type=_F32)
    acc += jnp.dot(t, Wi_[28:64, :], preferred_element_type=_F32)
    acc += jnp.dot(n, Wi_[64:76, :], preferred_element_type=_F32)
    acc += jnp.dot(c, Wi_[76:116, :], preferred_element_type=_F32)
    acc += jnp.dot(f, Wi_[116:128, :], preferred_element_type=_F32)
    x = lr(acc + bi[...])
    for hh in range(4):
        hcol = jnp.dot(x, W1[...][:, hh * 128:(hh + 1) * 128],
                       preferred_element_type=_F32)
        haug[hh, :, :] = hcol
        al[0, hh, :] = jnp.sum(hcol * as1[0, hh, :][None, :], axis=1)
        al[0, 4 + hh, :] = jnp.sum(hcol * ad1[0, hh, :][None, :], axis=1)


def _run_tc1(des, tweet, nump, catp, newf, Wd, bd, Wt, bt, Wn, bn, Wc, bc,
             Wf, bf, Wi, bi, W1, as1, ad1):
    full = lambda a: pl.BlockSpec(a.shape, lambda i: tuple(0 for _ in a.shape))
    row = lambda w: pl.BlockSpec((BN, w), lambda i: (i, 0))
    args = (des, tweet, nump, catp, newf, Wd, bd, Wt, bt, Wn, bn, Wc, bc,
            Wf, bf, Wi, bi, W1, as1, ad1)
    in_specs = [row(768), row(768), row(7), row(11), row(1)] + \
        [full(a) for a in args[5:]]
    haug, al = pl.pallas_call(
        _tc1_body,
        grid=(NP // BN,),
        in_specs=in_specs,
        out_specs=[pl.BlockSpec((4, BN, CF), lambda i: (0, i, 0)),
                   pl.BlockSpec((1, 8, BN), lambda i: (i, 0, 0))],
        out_shape=[jax.ShapeDtypeStruct((4, NP, CF), _F32),
                   jax.ShapeDtypeStruct((NP // BN, 8, BN), _F32)],
    )(*args)
    return haug, al


def _tc2_body(U1, den1, bg1, W2, as2, ad2, haug, al):
    W2_ = W2[...]
    acc = None
    for hh in range(4):
        den = jnp.maximum(den1[0, hh, :], 1e-30)[:, None]
        x1h = U1[hh, :, :] / den + bg1[...][hh * 128:(hh + 1) * 128][None, :]
        part = jnp.dot(x1h, W2_[hh * 128:(hh + 1) * 128, :],
                       preferred_element_type=_F32)
        acc = part if acc is None else acc + part
    haug[:, :] = acc
    al[0, 0, :] = jnp.sum(acc * as2[0, 0, :][None, :], axis=1)
    al[0, 1, :] = jnp.sum(acc * ad2[0, 0, :][None, :], axis=1)


def _run_tc2(U1, den1, bg1, W2, as2, ad2):
    full = lambda a: pl.BlockSpec(a.shape, lambda i: tuple(0 for _ in a.shape))
    haug, al = pl.pallas_call(
        _tc2_body,
        grid=(NP // BN,),
        in_specs=[pl.BlockSpec((4, BN, CF), lambda i: (0, i, 0)),
                  pl.BlockSpec((1, 4, BN), lambda i: (i, 0, 0)),
                  full(bg1), full(W2), full(as2), full(ad2)],
        out_specs=[pl.BlockSpec((BN, CF), lambda i: (i, 0)),
                   pl.BlockSpec((1, 2, BN), lambda i: (i, 0, 0))],
        out_shape=[jax.ShapeDtypeStruct((NP, CF), _F32),
                   jax.ShapeDtypeStruct((NP // BN, 2, BN), _F32)],
    )(U1, den1, bg1, W2, as2, ad2)
    return haug, al


def _tc3_body(U2, den2, bg2, Wo1, bo1, Wo2, bo2, out):
    lr = lambda v: jnp.where(v >= 0, v, 0.01 * v)
    Us = U2[0, :, :] + U2[1, :, :]
    den = jnp.maximum(den2[0, 0, :] + den2[0, 1, :], 1e-30)[:, None]
    x2 = Us / den + bg2[...][None, :]
    y = lr(jnp.dot(x2, Wo1[...], preferred_element_type=_F32) + bo1[...])
    out[...] = jnp.dot(y, Wo2[...], preferred_element_type=_F32) + bo2[...]


def _run_tc3(U2, den2, bg2, Wo1, bo1, Wo2, bo2):
    full = lambda a: pl.BlockSpec(a.shape, lambda i: tuple(0 for _ in a.shape))
    out = pl.pallas_call(
        _tc3_body,
        grid=(NP // BN,),
        in_specs=[pl.BlockSpec((2, BN, CF), lambda i: (0, i, 0)),
                  pl.BlockSpec((1, 2, BN), lambda i: (i, 0, 0)),
                  full(bg2), full(Wo1), full(bo1), full(Wo2), full(bo2)],
        out_specs=pl.BlockSpec((BN, 2), lambda i: (i, 0)),
        out_shape=jax.ShapeDtypeStruct((NP, 2), _F32),
    )(U2, den2, bg2, Wo1, bo1, Wo2, bo2)
    return out


def _sc_exa_body(H, src_h, dst_h, al_h, ex_h, als_v, ald_v, sbuf, dbuf, exbuf):
    c = lax.axis_index("c")
    s = lax.axis_index("s")
    iota = lax.iota(_I32, 16)
    per_tile = H * EP // 32
    wstart = (c * 16 + s) * per_tile
    hh = wstart // EP
    e0 = wstart - hh * EP
    pltpu.sync_copy(al_h.at[pl.ds(hh * NP, NP)], als_v)
    pltpu.sync_copy(al_h.at[pl.ds((H + hh) * NP, NP)], ald_v)

    def table_max(ref):
        def mbody(i, acc):
            v = plsc.load_gather(ref, [i * 16 + iota])
            return jnp.maximum(acc, v)
        acc = lax.fori_loop(0, NP // 16, mbody, jnp.full((16,), -1e30, _F32))
        return lax.reduce_max(acc, (0,))

    M = table_max(als_v) + table_max(ald_v)

    def chunk(ci, _):
        base = e0 + ci * KSLAB
        pltpu.sync_copy(src_h.at[pl.ds(base, KSLAB)], sbuf)
        pltpu.sync_copy(dst_h.at[pl.ds(base, KSLAB)], dbuf)

        def grp(j, _):
            w = j * 16 + iota
            s16 = plsc.load_gather(sbuf, [w])
            d16 = plsc.load_gather(dbuf, [w])
            a = plsc.load_gather(als_v, [s16]) + plsc.load_gather(ald_v, [d16])
            a = jnp.where(a >= 0, a, 0.2 * a)
            plsc.store_scatter(exbuf, [w], jnp.exp(a - M))
            return 0

        lax.fori_loop(0, KSLAB // 16, grp, 0)
        pltpu.sync_copy(exbuf, ex_h.at[pl.ds(hh * EP + base, KSLAB)])
        return 0

    lax.fori_loop(0, per_tile // KSLAB, chunk, 0)


def _run_sc_exa(H, src, dst, al_flat):
    mesh = plsc.VectorSubcoreMesh(core_axis_name="c", subcore_axis_name="s")
    k = pl.kernel(
        functools.partial(_sc_exa_body, H),
        out_type=jax.ShapeDtypeStruct((H * EP,), _F32),
        mesh=mesh,
        compiler_params=pltpu.CompilerParams(needs_layout_passes=False),
        scratch_types=[
            pltpu.VMEM((NP,), _F32),
            pltpu.VMEM((NP,), _F32),
            pltpu.VMEM((KSLAB,), _I32),
            pltpu.VMEM((KSLAB,), _I32),
            pltpu.VMEM((KSLAB,), _F32),
        ],
    )
    return k(src, dst, al_flat)


def _sc_agg_body(H, src_h, dst_h, ex_h, feat_h, z_h, U_h, den_h,
                 sbuf, dbuf2, exbuf, idxbuf, rows, den_v, dacc, dtmp,
                 U_sh, den_sh, gsem):
    c = lax.axis_index("c")
    s = lax.axis_index("s")
    iota = lax.iota(_I32, 16)
    lane0 = iota == 0
    n_phase = 2 if H == 4 else 1
    per_tile = EP // 16 if H == 4 else EP // 32
    nslab = per_tile // KSLAB
    zrows = NP // 16
    zf = jnp.zeros((16,), _F32)

    for ph in range(n_phase):
        if H == 4:
            hh = c * n_phase + ph
            e0 = s * per_tile
        else:
            hh = jnp.int32(0)
            e0 = c * (EP // 2) + s * per_tile
        upart = c * n_phase + ph
        pltpu.sync_copy(z_h.at[pl.ds(s * zrows, zrows)],
                        U_sh.at[pl.ds(s * zrows, zrows)])

        def dzero(j, _):
            plsc.store_scatter(den_v, [j * 16 + iota], zf)
            return 0

        lax.fori_loop(0, NP // 16, dzero, 0)
        plsc.subcore_barrier()

        def slab(ci, _):
            base = e0 + ci * KSLAB
            pltpu.sync_copy(src_h.at[pl.ds(base, KSLAB)], sbuf)
            pltpu.sync_copy(ex_h.at[pl.ds(hh * EP + base, KSLAB)], exbuf)

            def adj(j, _):
                w = j * 16 + iota
                s16 = plsc.load_gather(sbuf, [w])
                plsc.store_scatter(idxbuf, [w], s16 + hh * NP)
                return 0

            lax.fori_loop(0, KSLAB // 16, adj, 0)

            def sub(k, _):
                pltpu.async_copy(
                    feat_h.at[idxbuf.at[pl.ds(k * SB, SB)]],
                    rows, gsem).wait()
                for m in range(SB // 16):
                    pltpu.sync_copy(
                        dst_h.at[pl.ds(base + k * SB + m * 16, 16)],
                        dbuf2.at[m])

                def scale(e, _):
                    ev = plsc.load_gather(exbuf, [iota * 0 + (k * SB + e)])
                    row = iota * 0 + e
                    for j in range(CF // 16):
                        col = j * 16 + iota
                        v = plsc.load_gather(rows, [row, col])
                        plsc.store_scatter(rows, [row, col], v * ev)
                    return 0

                lax.fori_loop(0, SB, scale, 0)

                def dupd(j, _):
                    w = j * 16 + iota
                    d16 = plsc.load_gather(dbuf2, [w // 16, w - (w // 16) * 16])
                    e16 = plsc.load_gather(exbuf, [k * SB + w])
                    plsc.addupdate_scatter(den_v, [d16], e16)
                    return 0

                lax.fori_loop(0, SB // 16, dupd, 0)
                for m in range(SB // 16):
                    pltpu.sync_copy(rows.at[pl.ds(m * 16, 16)],
                                    U_sh.at[dbuf2.at[m]], add=True)
                return 0

            lax.fori_loop(0, KSLAB // SB, sub, 0)
            return 0

        lax.fori_loop(0, nslab, slab, 0)
        pltpu.sync_copy(den_v, den_sh.at[s])
        plsc.subcore_barrier()

        def dred0(j, _):
            plsc.store_scatter(dacc, [j * 16 + iota], zf)
            return 0

        lax.fori_loop(0, zrows // 16, dred0, 0)
        for t in range(16):
            pltpu.sync_copy(den_sh.at[t, pl.ds(s * zrows, zrows)], dtmp)

            def dred(j, _):
                w = j * 16 + iota
                a = plsc.load_gather(dacc, [w]) + plsc.load_gather(dtmp, [w])
                plsc.store_scatter(dacc, [w], a)
                return 0

            lax.fori_loop(0, zrows // 16, dred, 0)
        pltpu.sync_copy(U_sh.at[pl.ds(s * zrows, zrows)],
                        U_h.at[pl.ds(upart * NP + s * zrows, zrows)])
        pltpu.sync_copy(dacc, den_h.at[pl.ds(upart * NP + s * zrows, zrows)])
        plsc.subcore_barrier()


def _run_sc_agg(H, src, dst, ex, feat_flat, zeros_h):
    mesh = plsc.VectorSubcoreMesh(core_axis_name="c", subcore_axis_name="s")
    nout = 4 if H == 4 else 2
    k = pl.kernel(
        functools.partial(_sc_agg_body, H),
        out_type=[jax.ShapeDtypeStruct((nout * NP, CF), _F32),
                  jax.ShapeDtypeStruct((nout * NP,), _F32)],
        mesh=mesh,
        compiler_params=pltpu.CompilerParams(needs_layout_passes=False),
        scratch_types=[
            pltpu.VMEM((KSLAB,), _I32),
            pltpu.VMEM((SB // 16, 16), _I32),
            pltpu.VMEM((KSLAB,), _F32),
            pltpu.VMEM((KSLAB,), _I32),
            pltpu.VMEM((SB, CF), _F32),
            pltpu.VMEM((NP,), _F32),
            pltpu.VMEM((NP // 16,), _F32),
            pltpu.VMEM((NP // 16,), _F32),
            pltpu.VMEM_SHARED((NP, CF), _F32),
            pltpu.VMEM_SHARED((16, NP), _F32),
            pltpu.SemaphoreType.DMA,
        ],
    )
    return k(src, dst, ex, feat_flat, zeros_h)


def kernel(des, tweet, num_prop, cat_prop, new_feature, edge_index,
           Wd, bd, Wt, bt, Wn, bn, Wc, bc, Wf, bf, Wi, bi,
           W1, as1, ad1, bg1, W2, as2, ad2, bg2, Wo1, bo1, Wo2, bo2):
    padn = lambda a: jnp.pad(a, ((0, NP - NN), (0, 0)))
    des_p = padn(des)
    tweet_p = padn(tweet)
    nump_p = padn(num_prop)
    catp_p = padn(cat_prop)
    newf_p = padn(new_feature)

    ei = edge_index.astype(_I32)
    loop = jnp.arange(NN, dtype=_I32)
    fill = jnp.full((EP - EE - NN,), NN, _I32)
    src = jnp.concatenate([ei[0], loop, fill])
    dst = jnp.concatenate([ei[1], loop, fill])
    order = jnp.argsort(dst)
    perm = order.reshape(SB, EP // SB).T.reshape(-1)
    src = src[perm]
    dst = dst[perm]
    zeros_h = jnp.zeros((NP, CF), _F32)

    haug1, al1 = _run_tc1(des_p, tweet_p, nump_p, catp_p, newf_p,
                          Wd, bd, Wt, bt, Wn, bn, Wc, bc, Wf, bf,
                          Wi, bi, W1, as1, ad1)
    ex1 = _run_sc_exa(4, src, dst,
                      jnp.transpose(al1, (1, 0, 2)).reshape(8 * NP))
    U1, den1 = _run_sc_agg(4, src, dst, ex1, haug1.reshape(4 * NP, CF),
                           zeros_h)
    den1_t = jnp.transpose(den1.reshape(4, NP // BN, BN), (1, 0, 2))
    haug2, al2 = _run_tc2(U1.reshape(4, NP, CF), den1_t, bg1, W2, as2, ad2)
    ex2 = _run_sc_exa(1, src, dst,
                      jnp.transpose(al2, (1, 0, 2)).reshape(2 * NP))
    U2, den2 = _run_sc_agg(1, src, dst, ex2, haug2, zeros_h)
    den2_t = jnp.transpose(den2.reshape(2, NP // BN, BN), (1, 0, 2))
    out = _run_tc3(U2.reshape(2, NP, CF), den2_t, bg2, Wo1, bo1, Wo2, bo2)
    return out[:NN]


# speedup vs baseline: 12.0935x; 1.2816x over previous
import functools

import jax
import jax.numpy as jnp
from jax import lax
from jax.experimental import pallas as pl
from jax.experimental.pallas import tpu as pltpu
from jax.experimental.pallas import tpu_sc as plsc

NN = 10000
NP = 10240
EE = 320000
EP = 330240
BN = 1024
CF = 128
SB = 80
KSLAB = 3440

_F32 = jnp.float32
_I32 = jnp.int32


def _tc1_body(des, tweet, nump, catp, newf, Wd, bd, Wt, bt, Wn, bn, Wc, bc,
              Wf, bf, Wi, bi, W1, as1, ad1, haug, al):
    lr = lambda v: jnp.where(v >= 0, v, 0.01 * v)
    d = lr(jnp.dot(des[...], Wd[...], preferred_element_type=_F32) + bd[...])
    t = lr(jnp.dot(tweet[...], Wt[...], preferred_element_type=_F32) + bt[...])
    n = lr(jnp.dot(nump[...], Wn[...], preferred_element_type=_F32) + bn[...])
    c = lr(jnp.dot(catp[...], Wc[...], preferred_element_type=_F32) + bc[...])
    f = lr(jnp.dot(newf[...], Wf[...], preferred_element_type=_F32) + bf[...])
    Wi_ = Wi[...]
    acc = jnp.dot(d, Wi_[0:28, :], preferred_element_type=_F32)
    acc += jnp.dot(t, Wi_[28:64, :], preferred_element_type=_F32)
    acc += jnp.dot(n, Wi_[64:76, :], preferred_element_type=_F32)
    acc += jnp.dot(c, Wi_[76:116, :], preferred_element_type=_F32)
    acc += jnp.dot(f, Wi_[116:128, :], preferred_element_type=_F32)
    x = lr(acc + bi[...])
    for hh in range(4):
        hcol = jnp.dot(x, W1[...][:, hh * 128:(hh + 1) * 128],
                       preferred_element_type=_F32)
        haug[hh, :, :] = hcol
        al[0, hh, :] = jnp.sum(hcol * as1[0, hh, :][None, :], axis=1)
        al[0, 4 + hh, :] = jnp.sum(hcol * ad1[0, hh, :][None, :], axis=1)


def _run_tc1(des, tweet, nump, catp, newf, Wd, bd, Wt, bt, Wn, bn, Wc, bc,
             Wf, bf, Wi, bi, W1, as1, ad1):
    full = lambda a: pl.BlockSpec(a.shape, lambda i: tuple(0 for _ in a.shape))
    row = lambda w: pl.BlockSpec((BN, w), lambda i: (i, 0))
    args = (des, tweet, nump, catp, newf, Wd, bd, Wt, bt, Wn, bn, Wc, bc,
            Wf, bf, Wi, bi, W1, as1, ad1)
    in_specs = [row(768), row(768), row(7), row(11), row(1)] + \
        [full(a) for a in args[5:]]
    haug, al = pl.pallas_call(
        _tc1_body,
        grid=(NP // BN,),
        in_specs=in_specs,
        out_specs=[pl.BlockSpec((4, BN, CF), lambda i: (0, i, 0)),
                   pl.BlockSpec((1, 8, BN), lambda i: (i, 0, 0))],
        out_shape=[jax.ShapeDtypeStruct((4, NP, CF), _F32),
                   jax.ShapeDtypeStruct((NP // BN, 8, BN), _F32)],
    )(*args)
    return haug, al


def _tc2_body(U1, den1, bg1, W2, as2, ad2, haug, al):
    W2_ = W2[...]
    acc = None
    for hh in range(4):
        den = jnp.maximum(den1[0, hh, :], 1e-30)[:, None]
        x1h = U1[hh, :, :] / den + bg1[...][hh * 128:(hh + 1) * 128][None, :]
        part = jnp.dot(x1h, W2_[hh * 128:(hh + 1) * 128, :],
                       preferred_element_type=_F32)
        acc = part if acc is None else acc + part
    haug[:, :] = acc
    al[0, 0, :] = jnp.sum(acc * as2[0, 0, :][None, :], axis=1)
    al[0, 1, :] = jnp.sum(acc * ad2[0, 0, :][None, :], axis=1)


def _run_tc2(U1, den1, bg1, W2, as2, ad2):
    full = lambda a: pl.BlockSpec(a.shape, lambda i: tuple(0 for _ in a.shape))
    haug, al = pl.pallas_call(
        _tc2_body,
        grid=(NP // BN,),
        in_specs=[pl.BlockSpec((4, BN, CF), lambda i: (0, i, 0)),
                  pl.BlockSpec((1, 4, BN), lambda i: (i, 0, 0)),
                  full(bg1), full(W2), full(as2), full(ad2)],
        out_specs=[pl.BlockSpec((BN, CF), lambda i: (i, 0)),
                   pl.BlockSpec((1, 2, BN), lambda i: (i, 0, 0))],
        out_shape=[jax.ShapeDtypeStruct((NP, CF), _F32),
                   jax.ShapeDtypeStruct((NP // BN, 2, BN), _F32)],
    )(U1, den1, bg1, W2, as2, ad2)
    return haug, al


def _tc3_body(U2, den2, bg2, Wo1, bo1, Wo2, bo2, out):
    lr = lambda v: jnp.where(v >= 0, v, 0.01 * v)
    Us = U2[0, :, :] + U2[1, :, :]
    den = jnp.maximum(den2[0, 0, :] + den2[0, 1, :], 1e-30)[:, None]
    x2 = Us / den + bg2[...][None, :]
    y = lr(jnp.dot(x2, Wo1[...], preferred_element_type=_F32) + bo1[...])
    out[...] = jnp.dot(y, Wo2[...], preferred_element_type=_F32) + bo2[...]


def _run_tc3(U2, den2, bg2, Wo1, bo1, Wo2, bo2):
    full = lambda a: pl.BlockSpec(a.shape, lambda i: tuple(0 for _ in a.shape))
    out = pl.pallas_call(
        _tc3_body,
        grid=(NP // BN,),
        in_specs=[pl.BlockSpec((2, BN, CF), lambda i: (0, i, 0)),
                  pl.BlockSpec((1, 2, BN), lambda i: (i, 0, 0)),
                  full(bg2), full(Wo1), full(bo1), full(Wo2), full(bo2)],
        out_specs=pl.BlockSpec((BN, 2), lambda i: (i, 0)),
        out_shape=jax.ShapeDtypeStruct((NP, 2), _F32),
    )(U2, den2, bg2, Wo1, bo1, Wo2, bo2)
    return out


def _sc_exa_body(H, src_h, dst_h, al_h, ex_h, den_h,
                 als_v, ald_v, sbuf, dbuf, exbuf, den_v, dacc, dtmp, den_sh):
    c = lax.axis_index("c")
    s = lax.axis_index("s")
    iota = lax.iota(_I32, 16)
    per_tile = H * EP // 32
    wstart = (c * 16 + s) * per_tile
    hh = wstart // EP
    e0 = wstart - hh * EP
    pltpu.sync_copy(al_h.at[pl.ds(hh * NP, NP)], als_v)
    pltpu.sync_copy(al_h.at[pl.ds((H + hh) * NP, NP)], ald_v)
    zf = jnp.zeros((16,), _F32)

    def table_max(ref):
        def mbody(i, acc):
            v = plsc.load_gather(ref, [i * 16 + iota])
            return jnp.maximum(acc, v)
        acc = lax.fori_loop(0, NP // 16, mbody, jnp.full((16,), -1e30, _F32))
        return lax.reduce_max(acc, (0,))

    M = table_max(als_v) + table_max(ald_v)

    def dzero(j, _):
        plsc.store_scatter(den_v, [j * 16 + iota], zf)
        return 0

    lax.fori_loop(0, NP // 16, dzero, 0)

    def chunk(ci, _):
        base = e0 + ci * KSLAB
        pltpu.sync_copy(src_h.at[pl.ds(base, KSLAB)], sbuf)
        pltpu.sync_copy(dst_h.at[pl.ds(base, KSLAB)], dbuf)

        def grp(j, _):
            w = j * 16 + iota
            s16 = plsc.load_gather(sbuf, [w])
            d16 = plsc.load_gather(dbuf, [w])
            a = plsc.load_gather(als_v, [s16]) + plsc.load_gather(ald_v, [d16])
            a = jnp.where(a >= 0, a, 0.2 * a)
            exv = jnp.exp(a - M)
            plsc.store_scatter(exbuf, [w], exv)
            plsc.addupdate_scatter(den_v, [d16], exv)
            return 0

        lax.fori_loop(0, KSLAB // 16, grp, 0)
        pltpu.sync_copy(exbuf, ex_h.at[pl.ds(hh * EP + base, KSLAB)])
        return 0

    lax.fori_loop(0, per_tile // KSLAB, chunk, 0)

    pltpu.sync_copy(den_v, den_sh.at[s])
    plsc.subcore_barrier()
    nsl = NP // 16
    groups = [(range(0, 8), 2 * c), (range(8, 16), 2 * c + 1)] if H == 4 \
        else [(range(0, 16), c)]
    for rows_t, part in groups:
        def racc0(j, _):
            plsc.store_scatter(dacc, [j * 16 + iota], zf)
            return 0

        lax.fori_loop(0, nsl // 16, racc0, 0)
        for t in rows_t:
            pltpu.sync_copy(den_sh.at[t, pl.ds(s * nsl, nsl)],
                            dtmp.at[pl.ds(0, nsl)])

            def racc(j, _):
                w = j * 16 + iota
                a = plsc.load_gather(dacc, [w]) + plsc.load_gather(dtmp, [w])
                plsc.store_scatter(dacc, [w], a)
                return 0

            lax.fori_loop(0, nsl // 16, racc, 0)
        pltpu.sync_copy(dacc.at[pl.ds(0, nsl)],
                        den_h.at[pl.ds(part * NP + s * nsl, nsl)])


def _run_sc_exa(H, src, dst, al_flat):
    mesh = plsc.VectorSubcoreMesh(core_axis_name="c", subcore_axis_name="s")
    dnout = 4 if H == 4 else 2
    k = pl.kernel(
        functools.partial(_sc_exa_body, H),
        out_type=[jax.ShapeDtypeStruct((H * EP,), _F32),
                  jax.ShapeDtypeStruct((dnout * NP,), _F32)],
        mesh=mesh,
        compiler_params=pltpu.CompilerParams(needs_layout_passes=False),
        scratch_types=[
            pltpu.VMEM((NP,), _F32),
            pltpu.VMEM((NP,), _F32),
            pltpu.VMEM((KSLAB,), _I32),
            pltpu.VMEM((KSLAB,), _I32),
            pltpu.VMEM((KSLAB,), _F32),
            pltpu.VMEM((NP,), _F32),
            pltpu.VMEM((NP // 8,), _F32),
            pltpu.VMEM((NP // 8,), _F32),
            pltpu.VMEM_SHARED((16, NP), _F32),
        ],
    )
    return k(src, dst, al_flat)


def _sc_agg_body(H, src_h, dst_h, ex_h, feat_h, z_h, U_h,
                 sbuf, dbuf2, exbuf, idxbuf, rows, U_sh, gsem):
    c = lax.axis_index("c")
    s = lax.axis_index("s")
    iota = lax.iota(_I32, 16)
    n_phase = 2 if H == 4 else 1
    per_tile = EP // 16 if H == 4 else EP // 32
    nslab = per_tile // KSLAB
    zrows = NP // 16

    for ph in range(n_phase):
        if H == 4:
            hh = c * n_phase + ph
            e0 = s * per_tile
        else:
            hh = jnp.int32(0)
            e0 = c * (EP // 2) + s * per_tile
        upart = c * n_phase + ph
        pltpu.sync_copy(z_h.at[pl.ds(s * zrows, zrows)],
                        U_sh.at[pl.ds(s * zrows, zrows)])
        plsc.subcore_barrier()

        def slab(ci, _):
            base = e0 + ci * KSLAB
            pltpu.sync_copy(src_h.at[pl.ds(base, KSLAB)], sbuf)
            pltpu.sync_copy(ex_h.at[pl.ds(hh * EP + base, KSLAB)], exbuf)

            def adj(j, _):
                w = j * 16 + iota
                s16 = plsc.load_gather(sbuf, [w])
                plsc.store_scatter(idxbuf, [w], s16 + hh * NP)
                return 0

            lax.fori_loop(0, KSLAB // 16, adj, 0)

            def sub(k, _):
                pltpu.async_copy(
                    feat_h.at[idxbuf.at[pl.ds(k * SB, SB)]],
                    rows, gsem).wait()
                pltpu.sync_copy(dst_h.at[pl.ds(base + k * SB, SB)],
                                dbuf2.at[0])

                def scale(e, _):
                    ev = plsc.load_gather(exbuf, [iota * 0 + (k * SB + e)])
                    row = iota * 0 + e
                    for j in range(CF // 16):
                        col = j * 16 + iota
                        v = plsc.load_gather(rows, [row, col])
                        plsc.store_scatter(rows, [row, col], v * ev)
                    return 0

                lax.fori_loop(0, SB, scale, 0)
                pltpu.sync_copy(rows, U_sh.at[dbuf2.at[0]], add=True)
                return 0

            lax.fori_loop(0, KSLAB // SB, sub, 0)
            return 0

        lax.fori_loop(0, nslab, slab, 0)
        plsc.subcore_barrier()
        pltpu.sync_copy(U_sh.at[pl.ds(s * zrows, zrows)],
                        U_h.at[pl.ds(upart * NP + s * zrows, zrows)])
        plsc.subcore_barrier()


def _run_sc_agg(H, src, dst, ex, feat_flat, zeros_h):
    mesh = plsc.VectorSubcoreMesh(core_axis_name="c", subcore_axis_name="s")
    nout = 4 if H == 4 else 2
    k = pl.kernel(
        functools.partial(_sc_agg_body, H),
        out_type=jax.ShapeDtypeStruct((nout * NP, CF), _F32),
        mesh=mesh,
        compiler_params=pltpu.CompilerParams(needs_layout_passes=False),
        scratch_types=[
            pltpu.VMEM((KSLAB,), _I32),
            pltpu.VMEM((1, SB), _I32),
            pltpu.VMEM((KSLAB,), _F32),
            pltpu.VMEM((KSLAB,), _I32),
            pltpu.VMEM((SB, CF), _F32),
            pltpu.VMEM_SHARED((NP, CF), _F32),
            pltpu.SemaphoreType.DMA,
        ],
    )
    return k(src, dst, ex, feat_flat, zeros_h)


def kernel(des, tweet, num_prop, cat_prop, new_feature, edge_index,
           Wd, bd, Wt, bt, Wn, bn, Wc, bc, Wf, bf, Wi, bi,
           W1, as1, ad1, bg1, W2, as2, ad2, bg2, Wo1, bo1, Wo2, bo2):
    padn = lambda a: jnp.pad(a, ((0, NP - NN), (0, 0)))
    des_p = padn(des)
    tweet_p = padn(tweet)
    nump_p = padn(num_prop)
    catp_p = padn(cat_prop)
    newf_p = padn(new_feature)

    ei = edge_index.astype(_I32)
    loop = jnp.arange(NN, dtype=_I32)
    fill = jnp.full((EP - EE - NN,), NN, _I32)
    src = jnp.concatenate([ei[0], loop, fill])
    dst = jnp.concatenate([ei[1], loop, fill])
    order = jnp.argsort(dst)
    perm = order.reshape(SB, EP // SB).T.reshape(-1)
    src = src[perm]
    dst = dst[perm]
    zeros_h = jnp.zeros((NP, CF), _F32)

    haug1, al1 = _run_tc1(des_p, tweet_p, nump_p, catp_p, newf_p,
                          Wd, bd, Wt, bt, Wn, bn, Wc, bc, Wf, bf,
                          Wi, bi, W1, as1, ad1)
    ex1, den1 = _run_sc_exa(4, src, dst,
                            jnp.transpose(al1, (1, 0, 2)).reshape(8 * NP))
    U1 = _run_sc_agg(4, src, dst, ex1, haug1.reshape(4 * NP, CF), zeros_h)
    den1_t = jnp.transpose(den1.reshape(4, NP // BN, BN), (1, 0, 2))
    haug2, al2 = _run_tc2(U1.reshape(4, NP, CF), den1_t, bg1, W2, as2, ad2)
    ex2, den2 = _run_sc_exa(1, src, dst,
                            jnp.transpose(al2, (1, 0, 2)).reshape(2 * NP))
    U2 = _run_sc_agg(1, src, dst, ex2, haug2, zeros_h)
    den2_t = jnp.transpose(den2.reshape(2, NP // BN, BN), (1, 0, 2))
    out = _run_tc3(U2.reshape(2, NP, CF), den2_t, bg2, Wo1, bo1, Wo2, bo2)
    return out[:NN]
